# baseline (speedup 1.0000x reference)
import functools

import jax
import jax.numpy as jnp
from jax import lax
from jax.experimental import pallas as pl
from jax.experimental.pallas import tpu as pltpu
from jax.experimental.pallas import tpu_sc as plsc

_D = 256
_K = 8192
_TM = 256
_NC, _NS = 2, 16
_NW = _NC * _NS
_CH = 128


def _prep_body(e_ref, ebf_ref, en_ref):
    e = e_ref[...]
    ebf_ref[...] = e.astype(jnp.bfloat16)
    en_ref[...] = jnp.sum(e * e, axis=0, keepdims=True)


def _prep(embeddings):
    return pl.pallas_call(
        _prep_body,
        out_shape=(
            jax.ShapeDtypeStruct((_D, _K), jnp.bfloat16),
            jax.ShapeDtypeStruct((1, _K), jnp.float32),
        ),
    )(embeddings)


_PIECE = 2816


def _argmin_body(x_ref, ebf_ref, en_ref, o_ref):
    x = x_ref[...]
    xb2 = (x * 2.0).astype(jnp.bfloat16)
    sim2 = jnp.dot(xb2, ebf_ref[...], preferred_element_type=jnp.float32)
    xn = jnp.sum(x * x, axis=1, keepdims=True)
    d = (xn + en_ref[...]) - sim2
    acc = jnp.full((x.shape[0], 1), jnp.inf, jnp.float32)
    mstar = jnp.full((x.shape[0], 1), jnp.inf, jnp.float32)
    for j0 in range(0, _K, _PIECE):
        p = min(_PIECE, _K - j0)
        m = jnp.min(d[:, j0:j0 + p], axis=1, keepdims=True)
        take = m < acc
        mstar = jnp.where(take, m, mstar)
        acc = jnp.where(take, m.astype(jnp.bfloat16).astype(jnp.float32), acc)
    ii = lax.broadcasted_iota(jnp.int32, d.shape, 1)
    idx = jnp.min(jnp.where(d == mstar, ii, jnp.int32(_K)), axis=1)
    o_ref[0, 0, :] = idx


def _argmin(flat, ebf, enorm):
    m = flat.shape[0]
    nblk = m // _TM
    out = pl.pallas_call(
        _argmin_body,
        grid=(nblk,),
        in_specs=[
            pl.BlockSpec((_TM, _D), lambda i: (i, 0)),
            pl.BlockSpec((_D, _K), lambda i: (0, 0)),
            pl.BlockSpec((1, _K), lambda i: (0, 0)),
        ],
        out_specs=pl.BlockSpec((1, 1, _TM), lambda i: (i, 0, 0)),
        out_shape=jax.ShapeDtypeStruct((nblk, 1, _TM), jnp.int32),
        compiler_params=pltpu.CompilerParams(
            dimension_semantics=("parallel",)),
    )(flat, ebf, enorm)
    return out.reshape(m)


def _gather(table, idx):
    b = idx.shape[0]
    b_per_w = b // _NW
    mesh = plsc.VectorSubcoreMesh(core_axis_name="c", subcore_axis_name="s")

    @functools.partial(
        pl.kernel, mesh=mesh,
        out_type=jax.ShapeDtypeStruct((b, _D), jnp.float32),
        scratch_types=[
            pltpu.VMEM((_CH,), jnp.int32),
            pltpu.VMEM((_CH, _D), jnp.float32),
            pltpu.SemaphoreType.DMA,
        ],
    )
    def k(table_hbm, idx_hbm, out_hbm, idx_v, rows_v, sem):
        wid = lax.axis_index("s") * _NC + lax.axis_index("c")
        base = wid * b_per_w

        @pl.loop(0, b_per_w, step=_CH)
        def _(c):
            pltpu.sync_copy(idx_hbm.at[pl.ds(base + c, _CH)], idx_v)
            pltpu.async_copy(table_hbm.at[idx_v], rows_v, sem).wait()
            pltpu.sync_copy(rows_v, out_hbm.at[pl.ds(base + c, _CH)])

    return k(table, idx)


def kernel(x, embeddings):
    shape = x.shape
    flat = x.reshape(-1, _D)
    ebf, enorm = _prep(embeddings)
    idx = _argmin(flat, ebf, enorm)
    table = embeddings.T
    q = _gather(table, idx)
    return q.reshape(shape)


# speedup vs baseline: 1.1101x; 1.1101x over previous
import functools

import jax
import jax.numpy as jnp
from jax import lax
from jax.experimental import pallas as pl
from jax.experimental.pallas import tpu as pltpu
from jax.experimental.pallas import tpu_sc as plsc

_D = 256
_K = 8192
_TM = 512
_NC, _NS = 2, 16
_NW = _NC * _NS
_CH = 128


def _prep_body(e_ref, ebf_ref, en_ref):
    e = e_ref[...]
    ebf_ref[...] = e.astype(jnp.bfloat16)
    en_ref[...] = jnp.sum(e * e, axis=0, keepdims=True)


def _prep(embeddings):
    return pl.pallas_call(
        _prep_body,
        out_shape=(
            jax.ShapeDtypeStruct((_D, _K), jnp.bfloat16),
            jax.ShapeDtypeStruct((1, _K), jnp.float32),
        ),
    )(embeddings)


_PIECE = 2816


def _argmin_body(x_ref, ebf_ref, en_ref, o_ref):
    x = x_ref[...]
    xb = x.astype(jnp.bfloat16)
    sim = jnp.dot(xb, ebf_ref[...], preferred_element_type=jnp.float32)
    xn = jnp.sum(x * x, axis=1, keepdims=True)
    d = (xn + en_ref[...]) - 2.0 * sim
    acc = jnp.full((x.shape[0], 1), jnp.inf, jnp.float32)
    best = jnp.zeros((x.shape[0], 1), jnp.int32)
    for j0 in range(0, _K, _PIECE):
        p = min(_PIECE, _K - j0)
        blk = d[:, j0:j0 + p]
        m = jnp.min(blk, axis=1, keepdims=True)
        ii = lax.broadcasted_iota(jnp.int32, blk.shape, 1) + jnp.int32(j0)
        pidx = jnp.min(jnp.where(blk == m, ii, jnp.int32(_K)),
                       axis=1, keepdims=True)
        take = m < acc
        best = jnp.where(take, pidx, best)
        acc = jnp.where(take, m.astype(jnp.bfloat16).astype(jnp.float32), acc)
    o_ref[0, 0, :] = best[:, 0]


def _argmin(flat, ebf, enorm):
    m = flat.shape[0]
    nblk = m // _TM
    out = pl.pallas_call(
        _argmin_body,
        grid=(nblk,),
        in_specs=[
            pl.BlockSpec((_TM, _D), lambda i: (i, 0)),
            pl.BlockSpec((_D, _K), lambda i: (0, 0)),
            pl.BlockSpec((1, _K), lambda i: (0, 0)),
        ],
        out_specs=pl.BlockSpec((1, 1, _TM), lambda i: (i, 0, 0)),
        out_shape=jax.ShapeDtypeStruct((nblk, 1, _TM), jnp.int32),
        compiler_params=pltpu.CompilerParams(
            dimension_semantics=("parallel",)),
    )(flat, ebf, enorm)
    return out.reshape(m)


def _gather(table, idx):
    b = idx.shape[0]
    b_per_w = b // _NW
    mesh = plsc.VectorSubcoreMesh(core_axis_name="c", subcore_axis_name="s")

    @functools.partial(
        pl.kernel, mesh=mesh,
        out_type=jax.ShapeDtypeStruct((b, _D), jnp.float32),
        scratch_types=[
            pltpu.VMEM((_CH,), jnp.int32),
            pltpu.VMEM((_CH, _D), jnp.float32),
            pltpu.SemaphoreType.DMA,
        ],
    )
    def k(table_hbm, idx_hbm, out_hbm, idx_v, rows_v, sem):
        wid = lax.axis_index("s") * _NC + lax.axis_index("c")
        base = wid * b_per_w

        @pl.loop(0, b_per_w, step=_CH)
        def _(c):
            pltpu.sync_copy(idx_hbm.at[pl.ds(base + c, _CH)], idx_v)
            pltpu.async_copy(table_hbm.at[idx_v], rows_v, sem).wait()
            pltpu.sync_copy(rows_v, out_hbm.at[pl.ds(base + c, _CH)])

    return k(table, idx)


def kernel(x, embeddings):
    shape = x.shape
    flat = x.reshape(-1, _D)
    ebf, enorm = _prep(embeddings)
    idx = _argmin(flat, ebf, enorm)
    table = embeddings.T
    q = _gather(table, idx)
    return q.reshape(shape)


# speedup vs baseline: 1.1655x; 1.0499x over previous
import functools

import jax
import jax.numpy as jnp
from jax import lax
from jax.experimental import pallas as pl
from jax.experimental.pallas import tpu as pltpu
from jax.experimental.pallas import tpu_sc as plsc

_D = 256
_K = 8192
_TM = 1024
_NC, _NS = 2, 16
_NW = _NC * _NS
_CH = 128


def _prep_body(e_ref, ebf_ref, en_ref):
    e = e_ref[...]
    ebf_ref[...] = e.astype(jnp.bfloat16)
    en_ref[...] = jnp.sum(e * e, axis=0, keepdims=True)


def _prep(embeddings):
    return pl.pallas_call(
        _prep_body,
        out_shape=(
            jax.ShapeDtypeStruct((_D, _K), jnp.bfloat16),
            jax.ShapeDtypeStruct((1, _K), jnp.float32),
        ),
    )(embeddings)


_PIECE = 2816


def _argmin_body(x_ref, ebf_ref, en_ref, o_ref):
    x = x_ref[...]
    xb = x.astype(jnp.bfloat16)
    sim = jnp.dot(xb, ebf_ref[...], preferred_element_type=jnp.float32)
    xn = jnp.sum(x * x, axis=1, keepdims=True)
    d = (xn + en_ref[...]) - 2.0 * sim
    acc = jnp.full((x.shape[0], 1), jnp.inf, jnp.float32)
    best = jnp.zeros((x.shape[0], 1), jnp.int32)
    for j0 in range(0, _K, _PIECE):
        p = min(_PIECE, _K - j0)
        blk = d[:, j0:j0 + p]
        m = jnp.min(blk, axis=1, keepdims=True)
        ii = lax.broadcasted_iota(jnp.int32, blk.shape, 1) + jnp.int32(j0)
        pidx = jnp.min(jnp.where(blk == m, ii, jnp.int32(_K)),
                       axis=1, keepdims=True)
        take = m < acc
        best = jnp.where(take, pidx, best)
        acc = jnp.where(take, m.astype(jnp.bfloat16).astype(jnp.float32), acc)
    o_ref[0, 0, :] = best[:, 0]


def _argmin(flat, ebf, enorm):
    m = flat.shape[0]
    nblk = m // _TM
    out = pl.pallas_call(
        _argmin_body,
        grid=(nblk,),
        in_specs=[
            pl.BlockSpec((_TM, _D), lambda i: (i, 0)),
            pl.BlockSpec((_D, _K), lambda i: (0, 0)),
            pl.BlockSpec((1, _K), lambda i: (0, 0)),
        ],
        out_specs=pl.BlockSpec((1, 1, _TM), lambda i: (i, 0, 0)),
        out_shape=jax.ShapeDtypeStruct((nblk, 1, _TM), jnp.int32),
        compiler_params=pltpu.CompilerParams(
            dimension_semantics=("parallel",)),
    )(flat, ebf, enorm)
    return out.reshape(m)


def _gather(table, idx):
    b = idx.shape[0]
    b_per_w = b // _NW
    mesh = plsc.VectorSubcoreMesh(core_axis_name="c", subcore_axis_name="s")

    @functools.partial(
        pl.kernel, mesh=mesh,
        out_type=jax.ShapeDtypeStruct((b, _D), jnp.float32),
        scratch_types=[
            pltpu.VMEM((_CH,), jnp.int32),
            pltpu.VMEM((_CH, _D), jnp.float32),
            pltpu.SemaphoreType.DMA,
        ],
    )
    def k(table_hbm, idx_hbm, out_hbm, idx_v, rows_v, sem):
        wid = lax.axis_index("s") * _NC + lax.axis_index("c")
        base = wid * b_per_w

        @pl.loop(0, b_per_w, step=_CH)
        def _(c):
            pltpu.sync_copy(idx_hbm.at[pl.ds(base + c, _CH)], idx_v)
            pltpu.async_copy(table_hbm.at[idx_v], rows_v, sem).wait()
            pltpu.sync_copy(rows_v, out_hbm.at[pl.ds(base + c, _CH)])

    return k(table, idx)


def kernel(x, embeddings):
    shape = x.shape
    flat = x.reshape(-1, _D)
    ebf, enorm = _prep(embeddings)
    idx = _argmin(flat, ebf, enorm)
    table = embeddings.T
    q = _gather(table, idx)
    return q.reshape(shape)
